# baseline (speedup 1.0000x reference)
import functools

import jax
import jax.numpy as jnp
from jax import lax
from jax.experimental import pallas as pl
from jax.experimental.pallas import tpu as pltpu

_BF = jnp.bfloat16
_F32 = jnp.float32


def _xt_body(x_ref, o_ref):
    o_ref[...] = x_ref[...].astype(_BF).T


def _xt_call(x):
    n_tok, d = x.shape
    return pl.pallas_call(
        _xt_body,
        out_shape=jax.ShapeDtypeStruct((d, n_tok), _BF),
    )(x)


def _head_body(xt_ref, wt_ref, b_ref, o_ref):
    acc = jnp.dot(
        wt_ref[...].astype(_BF), xt_ref[...], preferred_element_type=_F32
    )
    o_ref[...] = acc + b_ref[...][:, None]


def _tail_body(xt_ref, p_ref, w_ref, b_ref, o_ref, h_ref, *, w_transposed, p_transposed):
    @pl.when(pl.program_id(0) == 0)
    def _():
        if p_transposed:
            h = jnp.dot(
                p_ref[...].astype(_BF), xt_ref[...], preferred_element_type=_F32
            )
        else:
            h = lax.dot_general(
                p_ref[...].astype(_BF),
                xt_ref[...],
                (((0,), (0,)), ((), ())),
                preferred_element_type=_F32,
            )
        h_ref[...] = h.astype(_BF)

    if w_transposed:
        acc = jnp.dot(w_ref[...].astype(_BF), h_ref[...], preferred_element_type=_F32)
    else:
        acc = lax.dot_general(
            w_ref[...].astype(_BF),
            h_ref[...],
            (((0,), (0,)), ((), ())),
            preferred_element_type=_F32,
        )
    o_ref[...] = acc + b_ref[...][:, None]


_C2_BN = 1024
_C2_NSLOT = 3


def _c2_body(nsteps, rem, xt_ref, p_ref, w_ref, b_ref, o_hbm, h_ref, obuf, sems):
    j = pl.program_id(0)
    last = nsteps - 1
    n_full = nsteps - (1 if rem else 0)
    slot = lax.rem(j, _C2_NSLOT)

    @pl.when(j == 0)
    def _():
        h_ref[...] = jnp.dot(
            p_ref[...].astype(_BF), xt_ref[...], preferred_element_type=_F32
        ).astype(_BF)

    @pl.when(j >= _C2_NSLOT)
    def _():
        pltpu.make_async_copy(
            obuf.at[slot],
            o_hbm.at[pl.ds((j - _C2_NSLOT) * _C2_BN, _C2_BN), :],
            sems.at[slot],
        ).wait()

    acc = lax.dot_general(
        w_ref[...].astype(_BF),
        h_ref[...],
        (((0,), (0,)), ((), ())),
        preferred_element_type=_F32,
    )
    obuf[slot] = acc + b_ref[...][:, None]

    @pl.when(j < n_full)
    def _():
        pltpu.make_async_copy(
            obuf.at[slot], o_hbm.at[pl.ds(j * _C2_BN, _C2_BN), :], sems.at[slot]
        ).start()

    if rem:

        @pl.when(j == last)
        def _():
            pltpu.make_async_copy(
                obuf.at[slot, :rem],
                o_hbm.at[pl.ds(n_full * _C2_BN, rem), :],
                sems.at[slot],
            ).start()

    @pl.when(j == last)
    def _():
        for k in range(_C2_NSLOT):
            sj = last - k
            if sj < 0:
                continue
            s = sj % _C2_NSLOT
            if rem and sj == last:
                pltpu.make_async_copy(
                    obuf.at[s, :rem],
                    o_hbm.at[pl.ds(n_full * _C2_BN, rem), :],
                    sems.at[s],
                ).wait()
            else:
                pltpu.make_async_copy(
                    obuf.at[s],
                    o_hbm.at[pl.ds(sj * _C2_BN, _C2_BN), :],
                    sems.at[s],
                ).wait()


def _c2_call(xt, pt, w, b):
    d, n_tok = xt.shape
    k, n_out = w.shape
    nsteps = pl.cdiv(n_out, _C2_BN)
    rem = n_out - (n_out // _C2_BN) * _C2_BN
    return pl.pallas_call(
        functools.partial(_c2_body, nsteps, rem),
        grid=(nsteps,),
        in_specs=[
            pl.BlockSpec((d, n_tok), lambda j: (0, 0)),
            pl.BlockSpec((k, d), lambda j: (0, 0)),
            pl.BlockSpec((k, _C2_BN), lambda j: (0, j)),
            pl.BlockSpec((_C2_BN,), lambda j: (j,)),
        ],
        out_specs=pl.BlockSpec(memory_space=pltpu.MemorySpace.HBM),
        out_shape=jax.ShapeDtypeStruct((n_out, n_tok), _F32),
        scratch_shapes=[
            pltpu.VMEM((k, n_tok), _BF),
            pltpu.VMEM((_C2_NSLOT, _C2_BN, n_tok), _F32),
            pltpu.SemaphoreType.DMA((_C2_NSLOT,)),
        ],
    )(xt, pt, w, b)


def _head_call(xt, wt, b, bn):
    d, n_tok = xt.shape
    n_out = wt.shape[0]
    return pl.pallas_call(
        _head_body,
        grid=(pl.cdiv(n_out, bn),),
        in_specs=[
            pl.BlockSpec((d, n_tok), lambda j: (0, 0)),
            pl.BlockSpec((bn, d), lambda j: (j, 0)),
            pl.BlockSpec((bn,), lambda j: (j,)),
        ],
        out_specs=pl.BlockSpec((bn, n_tok), lambda j: (j, 0)),
        out_shape=jax.ShapeDtypeStruct((n_out, n_tok), _F32),
    )(xt, wt, b)


def _tail_call(xt, p, w, b, bn, w_transposed, p_transposed=False):
    d, n_tok = xt.shape
    k = p.shape[0] if p_transposed else p.shape[1]
    n_out = w.shape[0] if w_transposed else w.shape[1]
    if w_transposed:
        w_spec = pl.BlockSpec((bn, k), lambda j: (j, 0))
    else:
        w_spec = pl.BlockSpec((k, bn), lambda j: (0, j))
    p_shape = (k, d) if p_transposed else (d, k)
    return pl.pallas_call(
        functools.partial(
            _tail_body, w_transposed=w_transposed, p_transposed=p_transposed
        ),
        grid=(pl.cdiv(n_out, bn),),
        in_specs=[
            pl.BlockSpec((d, n_tok), lambda j: (0, 0)),
            pl.BlockSpec(p_shape, lambda j: (0, 0)),
            w_spec,
            pl.BlockSpec((bn,), lambda j: (j,)),
        ],
        out_specs=pl.BlockSpec((bn, n_tok), lambda j: (j, 0)),
        out_shape=jax.ShapeDtypeStruct((n_out, n_tok), _F32),
        scratch_shapes=[pltpu.VMEM((k, n_tok), _BF)],
    )(xt, p, w, b)


def kernel(x, W0, b0, P1, W1, b1, P2, W2, b2):
    xt = _xt_call(x)
    lh = _head_call(xt, W0.T, b0, bn=512)
    lc1 = _tail_call(xt, P1, W1.T, b1, bn=1024, w_transposed=True)
    lc2 = _c2_call(xt, P2.T, W2, b2)
    return (lh.T, lc1.T, lc2.T)


# speedup vs baseline: 1.0212x; 1.0212x over previous
import functools

import jax
import jax.numpy as jnp
from jax import lax
from jax.experimental import pallas as pl

_BF = jnp.bfloat16
_F32 = jnp.float32


def _prep_body(x_ref, p1_ref, p2t_ref, xt_ref, h1_ref, h2_ref):
    xt = x_ref[...].astype(_BF).T
    xt_ref[...] = xt
    h1_ref[...] = lax.dot_general(
        p1_ref[...].astype(_BF),
        xt,
        (((0,), (0,)), ((), ())),
        preferred_element_type=_F32,
    ).astype(_BF)
    h2_ref[...] = jnp.dot(
        p2t_ref[...].astype(_BF), xt, preferred_element_type=_F32
    ).astype(_BF)


def _prep_call(x, p1, p2t):
    n_tok, d = x.shape
    k1 = p1.shape[1]
    k2 = p2t.shape[0]
    return pl.pallas_call(
        _prep_body,
        out_shape=(
            jax.ShapeDtypeStruct((d, n_tok), _BF),
            jax.ShapeDtypeStruct((k1, n_tok), _BF),
            jax.ShapeDtypeStruct((k2, n_tok), _BF),
        ),
    )(x, p1, p2t)


def _cluster_body(rhs_ref, w_ref, b_ref, o_ref, *, w_transposed):
    if w_transposed:
        acc = jnp.dot(
            w_ref[...].astype(_BF), rhs_ref[...], preferred_element_type=_F32
        )
    else:
        acc = lax.dot_general(
            w_ref[...].astype(_BF),
            rhs_ref[...],
            (((0,), (0,)), ((), ())),
            preferred_element_type=_F32,
        )
    o_ref[...] = acc + b_ref[...][:, None]


def _cluster_call(rhs, w, b, bn, w_transposed):
    k, n_tok = rhs.shape
    n_out = w.shape[0] if w_transposed else w.shape[1]
    if w_transposed:
        w_spec = pl.BlockSpec((bn, k), lambda j: (j, 0))
    else:
        w_spec = pl.BlockSpec((k, bn), lambda j: (0, j))
    return pl.pallas_call(
        functools.partial(_cluster_body, w_transposed=w_transposed),
        grid=(pl.cdiv(n_out, bn),),
        in_specs=[
            pl.BlockSpec((k, n_tok), lambda j: (0, 0)),
            w_spec,
            pl.BlockSpec((bn,), lambda j: (j,)),
        ],
        out_specs=pl.BlockSpec((bn, n_tok), lambda j: (j, 0)),
        out_shape=jax.ShapeDtypeStruct((n_out, n_tok), _F32),
    )(rhs, w, b)


def kernel(x, W0, b0, P1, W1, b1, P2, W2, b2):
    xt, h1t, h2t = _prep_call(x, P1, P2.T)
    lh = _cluster_call(xt, W0.T, b0, bn=512, w_transposed=True)
    lc1 = _cluster_call(h1t, W1.T, b1, bn=1024, w_transposed=True)
    lc2 = _cluster_call(h2t, W2, b2, bn=2048, w_transposed=False)
    return (lh.T, lc1.T, lc2.T)
